# Initial kernel scaffold; baseline (speedup 1.0000x reference)
import jax
import jax.numpy as jnp
from jax.experimental import pallas as pl


def kernel(query, support, symbol_emb_weight):
    raise NotImplementedError("write your pallas kernel here")


# speedup vs baseline: 2.7889x; 2.7889x over previous
import functools

import jax
import jax.numpy as jnp
from jax import lax
from jax.experimental import pallas as pl
from jax.experimental.pallas import tpu as pltpu
from jax.experimental.pallas import tpu_sc as plsc

NUM_Q = 4096
D = 128
PAIR = 2
DQ = D * PAIR
NCHUNK = DQ // 16
S_PAIRS = 128
S_ROWS = S_PAIRS * PAIR

_info = plsc.get_sparse_core_info()
NC, NS = _info.num_cores, _info.num_subcores
NW = NC * NS
QPW = NUM_Q // NW
RPW = QPW * PAIR


def _rsqrt(y):
    i = lax.bitcast_convert_type(y, jnp.int32)
    i = jnp.int32(0x5F3759DF) - lax.shift_right_logical(i, 1)
    r = lax.bitcast_convert_type(i, jnp.float32)
    for _ in range(3):
        r = r * (jnp.float32(1.5) - jnp.float32(0.5) * y * r * r)
    return r


def _build():
    mesh = plsc.VectorSubcoreMesh(core_axis_name="c", subcore_axis_name="s")

    @functools.partial(
        pl.kernel,
        mesh=mesh,
        out_type=jax.ShapeDtypeStruct((NUM_Q,), jnp.float32),
        scratch_types=[
            pltpu.VMEM((PAIR, RPW // PAIR), jnp.int32),
            pltpu.VMEM((PAIR, S_ROWS // PAIR), jnp.int32),
            pltpu.VMEM((S_ROWS, D), jnp.float32),
            pltpu.VMEM((RPW, D), jnp.float32),
            pltpu.VMEM((QPW,), jnp.float32),
            pltpu.VMEM((32,), jnp.float32),
            pltpu.SemaphoreType.DMA,
        ],
    )
    def sc_cosine(q_hbm, s_hbm, table_hbm, out_hbm,
                  qidx_v, sidx_v, srows_v, qrows_v, outv, red_v, sem):
        wid = lax.axis_index("s") * NC + lax.axis_index("c")

        pltpu.sync_copy(q_hbm.at[wid], qidx_v)
        pltpu.sync_copy(s_hbm, sidx_v)

        half = RPW // PAIR
        cps = []
        for j in range(PAIR):
            cps.append(pltpu.async_copy(
                table_hbm.at[sidx_v.at[j]],
                srows_v.at[pl.ds(j * half, half)], sem))
            cps.append(pltpu.async_copy(
                table_hbm.at[qidx_v.at[j]],
                qrows_v.at[pl.ds(j * half, half)], sem))
        for cp in cps:
            cp.wait()

        zero = jnp.zeros((16,), jnp.float32)

        def mean_body(k, acc):
            nxt = []
            for c in range(8):
                nxt.append(acc[c] + srows_v[2 * k, pl.ds(c * 16, 16)])
            for c in range(8):
                nxt.append(acc[8 + c] + srows_v[2 * k + 1, pl.ds(c * 16, 16)])
            return tuple(nxt)

        acc = lax.fori_loop(0, S_PAIRS, mean_body, (zero,) * NCHUNK)
        msc = [a * jnp.float32(1.0 / S_PAIRS) for a in acc]

        lane_iota = lax.iota(jnp.int32, 16)

        def lane_sum(v):
            t = v[0]
            for c in range(1, 16):
                t = t + v[c]
            return t

        tv = zero
        for c in range(NCHUNK):
            tv = tv + msc[c] * msc[c]
        sn2 = lane_sum(tv)

        def q_body(g, carry):
            numv = zero
            qnv = zero
            for j in range(16):
                i = g * 16 + j
                nv = zero
                qv = zero
                for c in range(NCHUNK):
                    row = 2 * i + (c // 8)
                    v = qrows_v[row, pl.ds((c % 8) * 16, 16)]
                    nv = nv + v * msc[c]
                    qv = qv + v * v
                sel = lane_iota == j
                numv = jnp.where(sel, lane_sum(nv), numv)
                qnv = jnp.where(sel, lane_sum(qv), qnv)
            y = jnp.maximum(qnv * sn2, jnp.float32(1e-16))
            outv[pl.ds(g * 16, 16)] = numv * _rsqrt(y)
            return carry

        lax.fori_loop(0, QPW // 16, q_body, 0)

        pltpu.sync_copy(outv, out_hbm.at[pl.ds(wid * QPW, QPW)])

    return sc_cosine


_sc_cosine = _build()


def kernel(query, support, symbol_emb_weight):
    q = query.astype(jnp.int32).reshape(NW, PAIR, RPW // PAIR)
    s = support.astype(jnp.int32).reshape(PAIR, S_ROWS // PAIR)
    return _sc_cosine(q, s, symbol_emb_weight)
